# baseline (device time: 21967 ns/iter reference)
import contextlib
import os

import jax
import jax.numpy as jnp
from jax import lax
from jax.experimental import pallas as pl
from jax.experimental.pallas import tpu as pltpu

_NO_COMM = os.environ.get("KNC") == "1"
_NO_COMPUTE = os.environ.get("KNP") == "1"

N_DEV = 4
B, SQ, SKV, HQ, DH = 2, 256, 256, 16, 64
H_PER = HQ // N_DEV
DM = 512
NC = 2
CH = SQ // NC
HF = CH // 2


def kernel(x, Wq, K_ext, V_ext, Wo):
    my_i = lax.axis_index("i")
    h0 = my_i * H_PER
    K_sh = lax.dynamic_slice_in_dim(K_ext, h0, H_PER, axis=2)
    V_sh = lax.dynamic_slice_in_dim(V_ext, h0, H_PER, axis=2)

    def body(x_ref, wq_ref, k_ref, v_ref, wo_ref, out_ref,
             comm_ref, send_sems, recv_sems):
        my_pos = lax.axis_index("i")
        pn = jnp.bitwise_xor(my_pos, 1)
        pd = jnp.bitwise_xor(my_pos, 2)
        bit = jnp.bitwise_and(my_pos, 1)

        def orows(c):
            return pl.ds(c * CH + bit * HF, HF)

        def xrows(c):
            return pl.ds(c * CH + (1 - bit) * HF, HF)

        barrier_sem = pltpu.get_barrier_semaphore()
        for p in (pn, pd):
            pl.semaphore_signal(barrier_sem, inc=1, device_id=(p,),
                                device_id_type=pl.DeviceIdType.MESH)

        xf = x_ref[...].astype(jnp.bfloat16).reshape(B * SQ, DM)
        q = jnp.dot(xf, wq_ref[...].astype(jnp.bfloat16),
                    preferred_element_type=jnp.float32)
        q = q.astype(jnp.bfloat16).reshape(B, SQ, H_PER, DH)
        wob = wo_ref[...].astype(jnp.bfloat16)

        qi = lax.broadcasted_iota(jnp.int32, (SQ, SKV), 0)
        ki = lax.broadcasted_iota(jnp.int32, (SQ, SKV), 1)
        mask = (jnp.abs(qi - ki) <= 128) | (ki < 32) | (qi < 32)

        def mk(stage, b, c):
            src_slot, partner, rows = (
                (0, pn, xrows(c)),
                (2, pd, orows(c)),
                (4, pn, orows(c)),
            )[stage]
            dst_slot = {0: 1, 2: 3, 4: 4}[src_slot]
            return pltpu.make_async_remote_copy(
                src_ref=comm_ref.at[src_slot, b, rows],
                dst_ref=comm_ref.at[dst_slot, b, rows],
                send_sem=send_sems.at[stage, b, c],
                recv_sem=recv_sems.at[stage, b, c],
                device_id=(partner,),
                device_id_type=pl.DeviceIdType.MESH,
            )

        rdmaA = [[mk(0, b, c) for c in range(NC)] for b in range(B)]
        rdmaB = [[mk(1, b, c) for c in range(NC)] for b in range(B)]
        rdmaC = [[mk(2, b, c) for c in range(NC)] for b in range(B)]

        barrier_waited = False
        for b in range(B):
            kb = k_ref[b].astype(jnp.bfloat16)
            vb = v_ref[b].astype(jnp.bfloat16)
            for c in range(NC):
                sl = pl.ds(c * CH, CH)
                if _NO_COMPUTE:
                    part = x_ref[b, sl].astype(jnp.bfloat16)
                else:
                    mrows = mask[c * CH:(c + 1) * CH]
                    ctx_heads = []
                    for h in range(H_PER):
                        s = lax.dot_general(
                            q[b, c * CH:(c + 1) * CH, h, :], kb[:, h, :],
                            (((1,), (1,)), ((), ())),
                            preferred_element_type=jnp.float32,
                        ) * 0.125
                        w = jnp.where(mrows, jnp.exp(s), 0.0)
                        recip = 1.0 / w.sum(axis=-1, keepdims=True)
                        ctx_heads.append(
                            jnp.dot(w.astype(jnp.bfloat16), vb[:, h, :],
                                    preferred_element_type=jnp.float32)
                            * recip)
                    ctx = jnp.concatenate(ctx_heads, axis=1).astype(jnp.bfloat16)
                    part = jnp.dot(ctx, wob,
                                   preferred_element_type=jnp.float32
                                   ).astype(jnp.bfloat16)
                comm_ref[0, b, sl] = part
                if not barrier_waited:
                    pl.semaphore_wait(barrier_sem, 2)
                    barrier_waited = True
                if not _NO_COMM:
                    rdmaA[b][c].start()

        if _NO_COMM:
            for b in range(B):
                out_ref[b] = comm_ref[0, b]
            return

        for b in range(B):
            for c in range(NC):
                ro = orows(c)
                rdmaA[b][c].wait_recv()
                comm_ref[2, b, ro] = comm_ref[0, b, ro] + comm_ref[1, b, ro]
                rdmaB[b][c].start()

        for b in range(B):
            for c in range(NC):
                ro = orows(c)
                rdmaB[b][c].wait_recv()
                full = comm_ref[2, b, ro] + comm_ref[3, b, ro]
                comm_ref[4, b, ro] = full
                out_ref[b, ro] = full
                rdmaC[b][c].start()

        for b in range(B):
            for c in range(NC):
                rx = xrows(c)
                rdmaC[b][c].wait_recv()
                out_ref[b, rx] = comm_ref[4, b, rx]

        for b in range(B):
            for c in range(NC):
                rdmaA[b][c].wait_send()
                rdmaB[b][c].wait_send()
                rdmaC[b][c].wait_send()

    out_shape = jax.ShapeDtypeStruct((B, SQ, DM), jnp.bfloat16)
    return pl.pallas_call(
        body,
        out_shape=out_shape,
        in_specs=[pl.BlockSpec(memory_space=pltpu.VMEM)] * 5,
        out_specs=pl.BlockSpec(memory_space=pltpu.VMEM),
        scratch_shapes=[
            pltpu.VMEM((5, B, SQ, DM), jnp.bfloat16),
            pltpu.SemaphoreType.DMA((3, B, NC)),
            pltpu.SemaphoreType.DMA((3, B, NC)),
        ],
        compiler_params=pltpu.CompilerParams(collective_id=0),
    )(x, Wq, K_sh, V_sh, Wo)
